# baseline (speedup 1.0000x reference)
import jax
import jax.numpy as jnp
from jax.experimental import pallas as pl
from jax.experimental.pallas import tpu as pltpu

BATCH = 16384
NF = 100
ED = 32
BB = 128


def _body(x_ref, m_ref, w_ref, bp_ref, mm_ref, out_ref):
    x = x_ref[...]
    m = m_ref[...]
    w = w_ref[...]
    bp = bp_ref[...]
    mm = mm_ref[...]
    x3 = x[:, :, None]
    m3 = m[:, :, None]
    t = x3 * w[None, :, :] + bp[None, :, :]
    out_ref[...] = t + m3 * (mm[None, :, :] - t)


def kernel(x_hat, mask, Wv, bv, missing_table, present_table):
    w = Wv[:, 0]
    w64 = jnp.concatenate([w, jnp.zeros((ED,), jnp.float32)]).reshape(1, 2 * ED)
    bp = jnp.concatenate(
        [jnp.broadcast_to(bv, (NF, ED)), present_table], axis=1)
    mm = jnp.concatenate(
        [jnp.zeros((NF, ED), jnp.float32), missing_table], axis=1)

    grid = (BATCH // BB,)
    out = pl.pallas_call(
        _body,
        grid=grid,
        in_specs=[
            pl.BlockSpec((BB, NF), lambda i: (i, 0)),
            pl.BlockSpec((BB, NF), lambda i: (i, 0)),
            pl.BlockSpec((1, 2 * ED), lambda i: (0, 0)),
            pl.BlockSpec((NF, 2 * ED), lambda i: (0, 0)),
            pl.BlockSpec((NF, 2 * ED), lambda i: (0, 0)),
        ],
        out_specs=pl.BlockSpec((BB, NF, 2 * ED), lambda i: (i, 0, 0)),
        out_shape=jax.ShapeDtypeStruct((BATCH, NF, 2 * ED), jnp.float32),
        compiler_params=pltpu.CompilerParams(
            dimension_semantics=("arbitrary",),
        ),
    )(x_hat, mask, w64, bp, mm)
    return out
